# Initial kernel scaffold; baseline (speedup 1.0000x reference)
import jax
import jax.numpy as jnp
from jax.experimental import pallas as pl


def kernel(node_attr, edge_index, edge_attr, edge_sh, fc_w1, fc_b1, fc_w2, fc_b2, ln_weight, ln_bias, ln_mean_shift):
    raise NotImplementedError("write your pallas kernel here")


# speedup vs baseline: 3.0104x; 3.0104x over previous
import functools

import numpy as np
import jax
import jax.numpy as jnp
from jax import lax
from jax.experimental import pallas as pl
from jax.experimental.pallas import tpu as pltpu
from jax.experimental.pallas import tpu_sc as plsc

N_NODES = 10000
N_EDGES = 40000
MUL_S = 48
MUL_V = 16
NODE_DIM = 96
SH_DIM = 4
N_EDGE_FEAT = 128
HIDDEN = 128
C_PATH = 0.125
EPS = 1e-5

NC = 2
NS = 16
NW = NC * NS
E_PAD = 40960
EPW = E_PAD // NW
CHUNK = 128
NCH = EPW // CHUNK
N_ACC = 10240
NPW = N_ACC // NS

BE = 512
BN = 512


def _sc_gather_body(table_hbm, idx_hbm, out_hbm, idx_v, rows_v, sem0, sem1):
    wid = lax.axis_index("s") * NC + lax.axis_index("c")
    sems = (sem0, sem1)
    pltpu.sync_copy(idx_hbm.at[wid], idx_v)
    cp = pltpu.async_copy(table_hbm.at[idx_v.at[0]], rows_v.at[0], sems[0])
    for j in range(NCH):
        cp.wait()
        if j + 1 < NCH:
            cp = pltpu.async_copy(table_hbm.at[idx_v.at[j + 1]],
                                  rows_v.at[(j + 1) % 2], sems[(j + 1) % 2])
        pltpu.sync_copy(rows_v.at[j % 2],
                        out_hbm.at[pl.ds(wid * EPW + j * CHUNK, CHUNK)])


def _sc_gather(node_attr_pad, idx3d):
    fn = pl.kernel(
        _sc_gather_body,
        out_type=jax.ShapeDtypeStruct((E_PAD, 128), jnp.float32),
        mesh=plsc.VectorSubcoreMesh(
            core_axis_name="c", subcore_axis_name="s", num_cores=NC,
            num_subcores=NS,
        ),
        scratch_types=[
            pltpu.VMEM((NCH, CHUNK), jnp.int32),
            pltpu.VMEM((2, CHUNK, 128), jnp.float32),
            pltpu.SemaphoreType.DMA,
            pltpu.SemaphoreType.DMA,
        ],
    )
    return fn(node_attr_pad, idx3d)


def _sc_scatter_body(tp_hbm, idx_hbm, z_hbm, out_hbm, idx_v, rows_v, acc_sp,
                     sem0, sem1):
    c = lax.axis_index("c")
    s = lax.axis_index("s")
    wid = s * NC + c
    sems = (sem0, sem1)
    pltpu.sync_copy(z_hbm, acc_sp.at[pl.ds(s * NPW, NPW)])
    pltpu.sync_copy(idx_hbm.at[wid], idx_v)
    cp = pltpu.async_copy(tp_hbm.at[pl.ds(wid * EPW, CHUNK)], rows_v.at[0],
                          sems[0])
    plsc.subcore_barrier()
    for j in range(NCH):
        cp.wait()
        if j + 1 < NCH:
            cp = pltpu.async_copy(
                tp_hbm.at[pl.ds(wid * EPW + (j + 1) * CHUNK, CHUNK)],
                rows_v.at[(j + 1) % 2], sems[(j + 1) % 2])
        pltpu.sync_copy(rows_v.at[j % 2], acc_sp.at[idx_v.at[j]], add=True)
    plsc.subcore_barrier()
    base = c * N_ACC + s * NPW
    pltpu.sync_copy(acc_sp.at[pl.ds(s * NPW, NPW)],
                    out_hbm.at[pl.ds(base, NPW)])


def _sc_scatter(tp, idx3d, z128):
    fn = pl.kernel(
        _sc_scatter_body,
        out_type=jax.ShapeDtypeStruct((NC * N_ACC, 128), jnp.float32),
        mesh=plsc.VectorSubcoreMesh(
            core_axis_name="c", subcore_axis_name="s", num_cores=NC,
            num_subcores=NS,
        ),
        scratch_types=[
            pltpu.VMEM((NCH, CHUNK), jnp.int32),
            pltpu.VMEM((2, CHUNK, 128), jnp.float32),
            pltpu.VMEM_SHARED((N_ACC, 128), jnp.float32),
            pltpu.SemaphoreType.DMA,
            pltpu.SemaphoreType.DMA,
        ],
    )
    return fn(tp, idx3d, z128)


def _tc_edge_body(ea_ref, sh_ref, xd_ref, w1_ref, b1_ref, gs_ref, bs_ref,
                  gv_ref, bv_ref, rs_ref, ss_ref, rv_ref, sv_ref, sel_ref,
                  q3_ref, p3_ref, o_ref):
    ea = ea_ref[...]
    sh = sh_ref[...]
    xd = xd_ref[:, :NODE_DIM]
    h = jax.nn.relu(jnp.dot(ea, w1_ref[...]) + b1_ref[0:1, :])
    ws = jnp.dot(h, gs_ref[...]) + bs_ref[0:1, :]
    wv = jnp.dot(h, gv_ref[...]) + bv_ref[0:1, :]
    xs = xd[:, :MUL_S]
    xvf = xd[:, MUL_S:]
    shs = sh[:, 0:1]
    shv = sh[:, 1:4]
    vv = jnp.concatenate([shv] * MUL_V, axis=1)
    bb = jnp.dot(xvf * vv, sel_ref[...])
    a_s = jnp.concatenate([xs * shs, bb], axis=1)
    u_s = jnp.dot(a_s, rs_ref[...])
    out_s = C_PATH * jnp.dot(ws * u_s, ss_ref[...])
    xvp = jnp.dot(xvf, q3_ref[...])
    ovs = []
    for j in range(3):
        a_vj = jnp.concatenate(
            [xs * shv[:, j:j + 1],
             xvp[:, j * MUL_V:(j + 1) * MUL_V] * shs], axis=1)
        u_vj = jnp.dot(a_vj, rv_ref[...])
        ovs.append(jnp.dot(wv * u_vj, sv_ref[...]))
    out_v = C_PATH * jnp.dot(jnp.concatenate(ovs, axis=1), p3_ref[...])
    n = out_s.shape[0]
    pad = jnp.concatenate(
        [jnp.ones((n, 1), jnp.float32), jnp.zeros((n, 31), jnp.float32)],
        axis=1)
    o_ref[...] = jnp.concatenate([out_s, out_v, pad], axis=1)


def _tc_edge(ea, sh, xd, w1, b1, gs, bs, gv, bv, rs, ss, rv, sv, sel, q3, p3):
    n_blk = E_PAD // BE
    full = lambda r, c: pl.BlockSpec((r, c), lambda i: (0, 0))
    out = pl.pallas_call(
        _tc_edge_body,
        grid=(n_blk,),
        in_specs=[
            pl.BlockSpec((BE, N_EDGE_FEAT), lambda i: (i, 0)),
            pl.BlockSpec((BE, SH_DIM), lambda i: (i, 0)),
            pl.BlockSpec((BE, 128), lambda i: (i, 0)),
            full(N_EDGE_FEAT, HIDDEN),
            full(8, HIDDEN),
            full(HIDDEN, 3072),
            full(8, 3072),
            full(HIDDEN, 1024),
            full(8, 1024),
            full(64, 3072),
            full(3072, MUL_S),
            full(64, 1024),
            full(1024, MUL_V),
            full(MUL_S, MUL_V),
            full(MUL_S, MUL_S),
            full(MUL_S, MUL_S),
        ],
        out_specs=pl.BlockSpec((BE, 128), lambda i: (i, 0)),
        out_shape=jax.ShapeDtypeStruct((E_PAD, 128), jnp.float32),
    )
    return out(ea, sh, xd, w1, b1, gs, bs, gv, bv, rs, ss, rv, sv, sel, q3, p3)


def _tc_ln_body(p0_ref, p1_ref, na_ref, lnc_ref, m2_ref, o_ref):
    psum = p0_ref[...] + p1_ref[...]
    ssum = psum[:, :NODE_DIM]
    cnt = psum[:, NODE_DIM:NODE_DIM + 1]
    x = ssum / jnp.maximum(cnt, 1.0) + na_ref[...]
    lnc = lnc_ref[...]
    w_s = lnc[0:1, :]
    b_s = lnc[1:2, :]
    ms_s = lnc[2:3, :]
    w_v = lnc[3:4, :]
    ms_v = lnc[4:5, :]
    f1 = x[:, :MUL_S]
    m1 = jnp.mean(f1, axis=1, keepdims=True)
    f1 = f1 - m1 * ms_s
    n1 = jnp.mean(f1 * f1, axis=1, keepdims=True)
    f1 = f1 * (lax.rsqrt(n1 + EPS) * w_s) + b_s
    x2 = x[:, MUL_S:]
    m2f = jnp.dot(x2, m2_ref[...])
    f2 = x2 - m2f * ms_v
    n2 = jnp.mean(f2 * f2, axis=1, keepdims=True)
    f2 = f2 * (lax.rsqrt(n2 + EPS) * w_v)
    o_ref[...] = jnp.concatenate([f1, f2], axis=1)


def _tc_ln(p0, p1, na_pad, lnc, m2c):
    n_blk = N_ACC // BN
    out = pl.pallas_call(
        _tc_ln_body,
        grid=(n_blk,),
        in_specs=[
            pl.BlockSpec((BN, 128), lambda i: (i, 0)),
            pl.BlockSpec((BN, 128), lambda i: (i, 0)),
            pl.BlockSpec((BN, NODE_DIM), lambda i: (i, 0)),
            pl.BlockSpec((8, MUL_S), lambda i: (0, 0)),
            pl.BlockSpec((MUL_S, MUL_S), lambda i: (0, 0)),
        ],
        out_specs=pl.BlockSpec((BN, NODE_DIM), lambda i: (i, 0)),
        out_shape=jax.ShapeDtypeStruct((N_ACC, NODE_DIM), jnp.float32),
    )
    return out(p0, p1, na_pad, lnc, m2c)


def _pad8(row):
    return np.pad(row[None, :], ((0, 7), (0, 0))).astype(np.float32)


_R_S = np.kron(np.eye(64), np.ones((1, MUL_S))).astype(np.float32)
_S_S = np.kron(np.ones((64, 1)), np.eye(MUL_S)).astype(np.float32)
_R_V = np.kron(np.eye(64), np.ones((1, MUL_V))).astype(np.float32)
_S_V = np.kron(np.ones((64, 1)), np.eye(MUL_V)).astype(np.float32)
_SEL = np.kron(np.eye(MUL_V), np.ones((3, 1))).astype(np.float32)
_Q3 = np.zeros((MUL_S, MUL_S), np.float32)
for _u in range(MUL_V):
    for _j in range(3):
        _Q3[3 * _u + _j, _j * MUL_V + _u] = 1.0
_P3 = np.zeros((MUL_S, MUL_S), np.float32)
for _w in range(MUL_V):
    for _j in range(3):
        _P3[_j * MUL_V + _w, 3 * _w + _j] = 1.0
_M2 = np.kron(np.ones((MUL_V, MUL_V), np.float32) / MUL_V,
              np.eye(3, dtype=np.float32))


def kernel(node_attr, edge_index, edge_attr, edge_sh, fc_w1, fc_b1, fc_w2,
           fc_b2, ln_weight, ln_bias, ln_mean_shift):
    f32 = jnp.float32
    edge_dst = jnp.concatenate(
        [edge_index[1], jnp.zeros((E_PAD - N_EDGES,), jnp.int32)])
    edge_src = jnp.concatenate(
        [edge_index[0], jnp.full((E_PAD - N_EDGES,), N_NODES, jnp.int32)])
    dst2d = edge_dst.reshape(NW, NCH, CHUNK)
    src2d = edge_src.reshape(NW, NCH, CHUNK)
    ea_pad = jnp.concatenate(
        [edge_attr, jnp.zeros((E_PAD - N_EDGES, N_EDGE_FEAT), f32)])
    sh_pad = jnp.concatenate(
        [edge_sh, jnp.zeros((E_PAD - N_EDGES, SH_DIM), f32)])
    na_pad = jnp.concatenate(
        [node_attr, jnp.zeros((N_ACC - N_NODES, NODE_DIM), f32)])
    na128 = jnp.concatenate(
        [node_attr, jnp.zeros((N_NODES, 128 - NODE_DIM), f32)], axis=1)

    g_s = jnp.concatenate([fc_w2[:, :2304], fc_w2[:, 3328:]], axis=1)
    g_v = fc_w2[:, 2304:3328]
    b_s = jnp.tile(jnp.concatenate([fc_b2[:2304], fc_b2[3328:]])[None, :],
                   (8, 1))
    b_v = jnp.tile(fc_b2[2304:3328][None, :], (8, 1))
    b1r = jnp.tile(fc_b1[None, :], (8, 1))

    lnc = jnp.zeros((8, MUL_S), f32)
    lnc = lnc.at[0].set(ln_weight[:MUL_S])
    lnc = lnc.at[1].set(ln_bias)
    lnc = lnc.at[2].set(ln_mean_shift[0, :MUL_S, 0])
    lnc = lnc.at[3].set(jnp.repeat(ln_weight[MUL_S:], 3))
    lnc = lnc.at[4].set(jnp.repeat(ln_mean_shift[0, MUL_S:, 0], 3))

    z128 = jnp.zeros((NPW, 128), f32)

    x_dst = _sc_gather(na128, dst2d)
    tp = _tc_edge(ea_pad, sh_pad, x_dst, fc_w1, b1r, g_s, b_s, g_v, b_v,
                  jnp.asarray(_R_S), jnp.asarray(_S_S), jnp.asarray(_R_V),
                  jnp.asarray(_S_V), jnp.asarray(_SEL), jnp.asarray(_Q3),
                  jnp.asarray(_P3))
    summed = _sc_scatter(tp, src2d, z128)
    summed = summed.reshape(NC, N_ACC, 128)
    out = _tc_ln(summed[0], summed[1], na_pad, lnc, jnp.asarray(_M2))
    return out[:N_NODES]


# speedup vs baseline: 3.0133x; 1.0010x over previous
import functools

import numpy as np
import jax
import jax.numpy as jnp
from jax import lax
from jax.experimental import pallas as pl
from jax.experimental.pallas import tpu as pltpu
from jax.experimental.pallas import tpu_sc as plsc

N_NODES = 10000
N_EDGES = 40000
MUL_S = 48
MUL_V = 16
NODE_DIM = 96
SH_DIM = 4
N_EDGE_FEAT = 128
HIDDEN = 128
C_PATH = 0.125
EPS = 1e-5

NC = 2
NS = 16
NW = NC * NS
E_PAD = 40960
EPW = E_PAD // NW
CHUNK = 128
NCH = EPW // CHUNK
N_ACC = 10240
NPW = N_ACC // NS

BE = 512
BN = 512


def _sc_gather_body(table_hbm, idx_hbm, out_hbm, idx_v, rows_v, sem0, sem1):
    wid = lax.axis_index("s") * NC + lax.axis_index("c")
    sems = (sem0, sem1)
    pltpu.sync_copy(idx_hbm.at[wid], idx_v)
    cp = pltpu.async_copy(table_hbm.at[idx_v.at[0]], rows_v.at[0], sems[0])
    for j in range(NCH):
        cp.wait()
        if j + 1 < NCH:
            cp = pltpu.async_copy(table_hbm.at[idx_v.at[j + 1]],
                                  rows_v.at[(j + 1) % 2], sems[(j + 1) % 2])
        pltpu.sync_copy(rows_v.at[j % 2],
                        out_hbm.at[pl.ds(wid * EPW + j * CHUNK, CHUNK)])


def _sc_gather(node_attr_pad, idx3d):
    fn = pl.kernel(
        _sc_gather_body,
        out_type=jax.ShapeDtypeStruct((E_PAD, 128), jnp.float32),
        mesh=plsc.VectorSubcoreMesh(
            core_axis_name="c", subcore_axis_name="s", num_cores=NC,
            num_subcores=NS,
        ),
        scratch_types=[
            pltpu.VMEM((NCH, CHUNK), jnp.int32),
            pltpu.VMEM((2, CHUNK, 128), jnp.float32),
            pltpu.SemaphoreType.DMA,
            pltpu.SemaphoreType.DMA,
        ],
    )
    return fn(node_attr_pad, idx3d)


def _sc_scatter_body(tp_hbm, idx_hbm, z_hbm, out_hbm, idx_v, rows_v, acc_sp,
                     sem0, sem1):
    c = lax.axis_index("c")
    s = lax.axis_index("s")
    wid = s * NC + c
    sems = (sem0, sem1)
    pltpu.sync_copy(z_hbm, acc_sp.at[pl.ds(s * NPW, NPW)])
    pltpu.sync_copy(idx_hbm.at[wid], idx_v)
    cp = pltpu.async_copy(tp_hbm.at[pl.ds(wid * EPW, CHUNK)], rows_v.at[0],
                          sems[0])
    plsc.subcore_barrier()
    for j in range(NCH):
        cp.wait()
        if j + 1 < NCH:
            cp = pltpu.async_copy(
                tp_hbm.at[pl.ds(wid * EPW + (j + 1) * CHUNK, CHUNK)],
                rows_v.at[(j + 1) % 2], sems[(j + 1) % 2])
        pltpu.sync_copy(rows_v.at[j % 2], acc_sp.at[idx_v.at[j]], add=True)
    plsc.subcore_barrier()
    base = c * N_ACC + s * NPW
    pltpu.sync_copy(acc_sp.at[pl.ds(s * NPW, NPW)],
                    out_hbm.at[pl.ds(base, NPW)])


def _sc_scatter(tp, idx3d, z128):
    fn = pl.kernel(
        _sc_scatter_body,
        out_type=jax.ShapeDtypeStruct((NC * N_ACC, 128), jnp.float32),
        mesh=plsc.VectorSubcoreMesh(
            core_axis_name="c", subcore_axis_name="s", num_cores=NC,
            num_subcores=NS,
        ),
        scratch_types=[
            pltpu.VMEM((NCH, CHUNK), jnp.int32),
            pltpu.VMEM((2, CHUNK, 128), jnp.float32),
            pltpu.VMEM_SHARED((N_ACC, 128), jnp.float32),
            pltpu.SemaphoreType.DMA,
            pltpu.SemaphoreType.DMA,
        ],
    )
    return fn(tp, idx3d, z128)


def _tc_edge_body(ea_ref, sh_ref, xd_ref, w1_ref, b1_ref, gs_ref, bs_ref,
                  gv_ref, bv_ref, rs_ref, ss_ref, rv_ref, sv_ref, sel_ref,
                  q3_ref, p3_ref, o_ref):
    f32 = jnp.float32
    bf16 = jnp.bfloat16
    ea = ea_ref[...]
    sh = sh_ref[...]
    xd = xd_ref[:, :NODE_DIM]
    h = jax.nn.relu(
        jnp.dot(ea, w1_ref[...], preferred_element_type=f32)
        + b1_ref[0:1, :])
    hb = h.astype(bf16)
    ws = (jnp.dot(hb, gs_ref[...], preferred_element_type=f32)
          + bs_ref[0:1, :])
    wv = (jnp.dot(hb, gv_ref[...], preferred_element_type=f32)
          + bv_ref[0:1, :])
    xs = xd[:, :MUL_S]
    xvf = xd[:, MUL_S:]
    shs = sh[:, 0:1]
    shv = sh[:, 1:4]
    vv = jnp.concatenate([shv] * MUL_V, axis=1)
    bb = jnp.dot(xvf * vv, sel_ref[...])
    a_s = jnp.concatenate([xs * shs, bb], axis=1).astype(bf16)
    u_s = jnp.dot(a_s, rs_ref[...], preferred_element_type=f32)
    out_s = C_PATH * jnp.dot((ws * u_s).astype(bf16), ss_ref[...],
                             preferred_element_type=f32)
    xvp = jnp.dot(xvf, q3_ref[...])
    ovs = []
    for j in range(3):
        a_vj = jnp.concatenate(
            [xs * shv[:, j:j + 1],
             xvp[:, j * MUL_V:(j + 1) * MUL_V] * shs],
            axis=1).astype(bf16)
        u_vj = jnp.dot(a_vj, rv_ref[...], preferred_element_type=f32)
        ovs.append(jnp.dot((wv * u_vj).astype(bf16), sv_ref[...],
                           preferred_element_type=f32))
    out_v = C_PATH * jnp.dot(jnp.concatenate(ovs, axis=1), p3_ref[...])
    n = out_s.shape[0]
    pad = jnp.concatenate(
        [jnp.ones((n, 1), jnp.float32), jnp.zeros((n, 31), jnp.float32)],
        axis=1)
    o_ref[...] = jnp.concatenate([out_s, out_v, pad], axis=1)


def _tc_edge(ea, sh, xd, w1, b1, gs, bs, gv, bv, rs, ss, rv, sv, sel, q3, p3):
    n_blk = E_PAD // BE
    full = lambda r, c: pl.BlockSpec((r, c), lambda i: (0, 0))
    out = pl.pallas_call(
        _tc_edge_body,
        grid=(n_blk,),
        in_specs=[
            pl.BlockSpec((BE, N_EDGE_FEAT), lambda i: (i, 0)),
            pl.BlockSpec((BE, SH_DIM), lambda i: (i, 0)),
            pl.BlockSpec((BE, 128), lambda i: (i, 0)),
            full(N_EDGE_FEAT, HIDDEN),
            full(8, HIDDEN),
            full(HIDDEN, 3072),
            full(8, 3072),
            full(HIDDEN, 1024),
            full(8, 1024),
            full(64, 3072),
            full(3072, MUL_S),
            full(64, 1024),
            full(1024, MUL_V),
            full(MUL_S, MUL_V),
            full(MUL_S, MUL_S),
            full(MUL_S, MUL_S),
        ],
        out_specs=pl.BlockSpec((BE, 128), lambda i: (i, 0)),
        out_shape=jax.ShapeDtypeStruct((E_PAD, 128), jnp.float32),
    )
    return out(ea, sh, xd, w1, b1, gs, bs, gv, bv, rs, ss, rv, sv, sel, q3, p3)


def _tc_ln_body(p0_ref, p1_ref, na_ref, lnc_ref, m2_ref, o_ref):
    psum = p0_ref[...] + p1_ref[...]
    ssum = psum[:, :NODE_DIM]
    cnt = psum[:, NODE_DIM:NODE_DIM + 1]
    x = ssum / jnp.maximum(cnt, 1.0) + na_ref[...]
    lnc = lnc_ref[...]
    w_s = lnc[0:1, :]
    b_s = lnc[1:2, :]
    ms_s = lnc[2:3, :]
    w_v = lnc[3:4, :]
    ms_v = lnc[4:5, :]
    f1 = x[:, :MUL_S]
    m1 = jnp.mean(f1, axis=1, keepdims=True)
    f1 = f1 - m1 * ms_s
    n1 = jnp.mean(f1 * f1, axis=1, keepdims=True)
    f1 = f1 * (lax.rsqrt(n1 + EPS) * w_s) + b_s
    x2 = x[:, MUL_S:]
    m2f = jnp.dot(x2, m2_ref[...])
    f2 = x2 - m2f * ms_v
    n2 = jnp.mean(f2 * f2, axis=1, keepdims=True)
    f2 = f2 * (lax.rsqrt(n2 + EPS) * w_v)
    o_ref[...] = jnp.concatenate([f1, f2], axis=1)


def _tc_ln(p0, p1, na_pad, lnc, m2c):
    n_blk = N_ACC // BN
    out = pl.pallas_call(
        _tc_ln_body,
        grid=(n_blk,),
        in_specs=[
            pl.BlockSpec((BN, 128), lambda i: (i, 0)),
            pl.BlockSpec((BN, 128), lambda i: (i, 0)),
            pl.BlockSpec((BN, NODE_DIM), lambda i: (i, 0)),
            pl.BlockSpec((8, MUL_S), lambda i: (0, 0)),
            pl.BlockSpec((MUL_S, MUL_S), lambda i: (0, 0)),
        ],
        out_specs=pl.BlockSpec((BN, NODE_DIM), lambda i: (i, 0)),
        out_shape=jax.ShapeDtypeStruct((N_ACC, NODE_DIM), jnp.float32),
    )
    return out(p0, p1, na_pad, lnc, m2c)


def _pad8(row):
    return np.pad(row[None, :], ((0, 7), (0, 0))).astype(np.float32)


_R_S = np.kron(np.eye(64), np.ones((1, MUL_S))).astype(np.float32)
_S_S = np.kron(np.ones((64, 1)), np.eye(MUL_S)).astype(np.float32)
_R_V = np.kron(np.eye(64), np.ones((1, MUL_V))).astype(np.float32)
_S_V = np.kron(np.ones((64, 1)), np.eye(MUL_V)).astype(np.float32)
_SEL = np.kron(np.eye(MUL_V), np.ones((3, 1))).astype(np.float32)
_Q3 = np.zeros((MUL_S, MUL_S), np.float32)
for _u in range(MUL_V):
    for _j in range(3):
        _Q3[3 * _u + _j, _j * MUL_V + _u] = 1.0
_P3 = np.zeros((MUL_S, MUL_S), np.float32)
for _w in range(MUL_V):
    for _j in range(3):
        _P3[_j * MUL_V + _w, 3 * _w + _j] = 1.0
_M2 = np.kron(np.ones((MUL_V, MUL_V), np.float32) / MUL_V,
              np.eye(3, dtype=np.float32))


def kernel(node_attr, edge_index, edge_attr, edge_sh, fc_w1, fc_b1, fc_w2,
           fc_b2, ln_weight, ln_bias, ln_mean_shift):
    f32 = jnp.float32
    edge_dst = jnp.concatenate(
        [edge_index[1], jnp.zeros((E_PAD - N_EDGES,), jnp.int32)])
    edge_src = jnp.concatenate(
        [edge_index[0], jnp.full((E_PAD - N_EDGES,), N_NODES, jnp.int32)])
    dst2d = edge_dst.reshape(NW, NCH, CHUNK)
    src2d = edge_src.reshape(NW, NCH, CHUNK)
    ea_pad = jnp.concatenate(
        [edge_attr, jnp.zeros((E_PAD - N_EDGES, N_EDGE_FEAT), f32)])
    sh_pad = jnp.concatenate(
        [edge_sh, jnp.zeros((E_PAD - N_EDGES, SH_DIM), f32)])
    na_pad = jnp.concatenate(
        [node_attr, jnp.zeros((N_ACC - N_NODES, NODE_DIM), f32)])
    na128 = jnp.concatenate(
        [node_attr, jnp.zeros((N_NODES, 128 - NODE_DIM), f32)], axis=1)

    bf16 = jnp.bfloat16
    g_s = jnp.concatenate([fc_w2[:, :2304], fc_w2[:, 3328:]],
                          axis=1).astype(bf16)
    g_v = fc_w2[:, 2304:3328].astype(bf16)
    b_s = jnp.tile(jnp.concatenate([fc_b2[:2304], fc_b2[3328:]])[None, :],
                   (8, 1)).astype(bf16)
    b_v = jnp.tile(fc_b2[2304:3328][None, :], (8, 1)).astype(bf16)
    b1r = jnp.tile(fc_b1[None, :], (8, 1))

    lnc = jnp.zeros((8, MUL_S), f32)
    lnc = lnc.at[0].set(ln_weight[:MUL_S])
    lnc = lnc.at[1].set(ln_bias)
    lnc = lnc.at[2].set(ln_mean_shift[0, :MUL_S, 0])
    lnc = lnc.at[3].set(jnp.repeat(ln_weight[MUL_S:], 3))
    lnc = lnc.at[4].set(jnp.repeat(ln_mean_shift[0, MUL_S:, 0], 3))

    z128 = jnp.zeros((NPW, 128), f32)

    x_dst = _sc_gather(na128, dst2d)
    tp = _tc_edge(ea_pad.astype(bf16), sh_pad, x_dst, fc_w1.astype(bf16),
                  b1r, g_s, b_s, g_v, b_v,
                  jnp.asarray(_R_S, bf16), jnp.asarray(_S_S, bf16),
                  jnp.asarray(_R_V, bf16), jnp.asarray(_S_V, bf16),
                  jnp.asarray(_SEL), jnp.asarray(_Q3), jnp.asarray(_P3))
    summed = _sc_scatter(tp, src2d, z128)
    summed = summed.reshape(NC, N_ACC, 128)
    out = _tc_ln(summed[0], summed[1], na_pad, lnc, jnp.asarray(_M2))
    return out[:N_NODES]


# speedup vs baseline: 3.4589x; 1.1479x over previous
import functools

import numpy as np
import jax
import jax.numpy as jnp
from jax import lax
from jax.experimental import pallas as pl
from jax.experimental.pallas import tpu as pltpu
from jax.experimental.pallas import tpu_sc as plsc

N_NODES = 10000
N_EDGES = 40000
MUL_S = 48
MUL_V = 16
NODE_DIM = 96
SH_DIM = 4
N_EDGE_FEAT = 128
HIDDEN = 128
C_PATH = 0.125
EPS = 1e-5

NC = 2
NS = 16
NW = NC * NS
E_PAD = 40960
EPW = E_PAD // NW
CHUNK = 128
NCH = EPW // CHUNK
N_ACC = 10240
NPW = N_ACC // NS

BE = 512
BN = 512


def _sc_gather_body(table_hbm, idx_hbm, out_hbm, idx_v, rows_v, sem0, sem1):
    wid = lax.axis_index("s") * NC + lax.axis_index("c")
    sems = (sem0, sem1)
    pltpu.sync_copy(idx_hbm.at[wid], idx_v)
    cp = pltpu.async_copy(table_hbm.at[idx_v.at[0]], rows_v.at[0], sems[0])
    for j in range(NCH):
        cp.wait()
        if j + 1 < NCH:
            cp = pltpu.async_copy(table_hbm.at[idx_v.at[j + 1]],
                                  rows_v.at[(j + 1) % 2], sems[(j + 1) % 2])
        pltpu.sync_copy(rows_v.at[j % 2],
                        out_hbm.at[pl.ds(wid * EPW + j * CHUNK, CHUNK)])


def _sc_gather(node_attr_pad, idx3d):
    fn = pl.kernel(
        _sc_gather_body,
        out_type=jax.ShapeDtypeStruct((E_PAD, 128), jnp.float32),
        mesh=plsc.VectorSubcoreMesh(
            core_axis_name="c", subcore_axis_name="s", num_cores=NC,
            num_subcores=NS,
        ),
        scratch_types=[
            pltpu.VMEM((NCH, CHUNK), jnp.int32),
            pltpu.VMEM((2, CHUNK, 128), jnp.float32),
            pltpu.SemaphoreType.DMA,
            pltpu.SemaphoreType.DMA,
        ],
    )
    return fn(node_attr_pad, idx3d)


def _sc_scatter_body(tp_hbm, idx_hbm, z_hbm, out_hbm, idx_v, rows_v, acc_sp,
                     sem0, sem1):
    c = lax.axis_index("c")
    s = lax.axis_index("s")
    wid = s * NC + c
    sems = (sem0, sem1)
    pltpu.sync_copy(z_hbm, acc_sp.at[pl.ds(s * NPW, NPW)])
    pltpu.sync_copy(idx_hbm.at[wid], idx_v)
    cp = pltpu.async_copy(tp_hbm.at[pl.ds(wid * EPW, CHUNK)], rows_v.at[0],
                          sems[0])
    plsc.subcore_barrier()
    for j in range(NCH):
        cp.wait()
        if j + 1 < NCH:
            cp = pltpu.async_copy(
                tp_hbm.at[pl.ds(wid * EPW + (j + 1) * CHUNK, CHUNK)],
                rows_v.at[(j + 1) % 2], sems[(j + 1) % 2])
        pltpu.sync_copy(rows_v.at[j % 2], acc_sp.at[idx_v.at[j]], add=True)
    plsc.subcore_barrier()
    base = c * N_ACC + s * NPW
    pltpu.sync_copy(acc_sp.at[pl.ds(s * NPW, NPW)],
                    out_hbm.at[pl.ds(base, NPW)])


def _sc_scatter(tp, idx3d, z128):
    fn = pl.kernel(
        _sc_scatter_body,
        out_type=jax.ShapeDtypeStruct((NC * N_ACC, 128), jnp.float32),
        mesh=plsc.VectorSubcoreMesh(
            core_axis_name="c", subcore_axis_name="s", num_cores=NC,
            num_subcores=NS,
        ),
        scratch_types=[
            pltpu.VMEM((NCH, CHUNK), jnp.int32),
            pltpu.VMEM((2, CHUNK, 128), jnp.float32),
            pltpu.VMEM_SHARED((N_ACC, 128), jnp.float32),
            pltpu.SemaphoreType.DMA,
            pltpu.SemaphoreType.DMA,
        ],
    )
    return fn(tp, idx3d, z128)


def _treesum64(p):
    w = p.shape[1]
    target = w // 64
    while w > target:
        w //= 2
        p = p[:, :w] + p[:, w:2 * w]
    return p


def _tc_edge_body(ea_ref, sh_ref, xd_ref, w1_ref, b1_ref, gs_ref, bs_ref,
                  gv_ref, bv_ref, rs_ref, rv_ref, sel_ref,
                  q3_ref, p3_ref, o_ref):
    f32 = jnp.float32
    bf16 = jnp.bfloat16
    ea = ea_ref[...]
    sh = sh_ref[...]
    xd = xd_ref[:, :NODE_DIM]
    h = jax.nn.relu(
        jnp.dot(ea, w1_ref[...], preferred_element_type=f32)
        + b1_ref[0:1, :])
    hb = h.astype(bf16)
    ws = (jnp.dot(hb, gs_ref[...], preferred_element_type=f32)
          + bs_ref[0:1, :])
    wv = (jnp.dot(hb, gv_ref[...], preferred_element_type=f32)
          + bv_ref[0:1, :])
    xs = xd[:, :MUL_S]
    xvf = xd[:, MUL_S:]
    shs = sh[:, 0:1]
    shv = sh[:, 1:4]
    vv = jnp.concatenate([shv] * MUL_V, axis=1)
    bb = jnp.dot(xvf * vv, sel_ref[...])
    a_s = jnp.concatenate([xs * shs, bb], axis=1).astype(bf16)
    u_s = jnp.dot(a_s, rs_ref[...], preferred_element_type=f32)
    out_s = C_PATH * _treesum64(ws * u_s)
    xvp = jnp.dot(xvf, q3_ref[...])
    ovs = []
    for j in range(3):
        a_vj = jnp.concatenate(
            [xs * shv[:, j:j + 1],
             xvp[:, j * MUL_V:(j + 1) * MUL_V] * shs],
            axis=1).astype(bf16)
        u_vj = jnp.dot(a_vj, rv_ref[...], preferred_element_type=f32)
        ovs.append(_treesum64(wv * u_vj))
    out_v = C_PATH * jnp.dot(jnp.concatenate(ovs, axis=1), p3_ref[...])
    n = out_s.shape[0]
    pad = jnp.concatenate(
        [jnp.ones((n, 1), jnp.float32), jnp.zeros((n, 31), jnp.float32)],
        axis=1)
    o_ref[...] = jnp.concatenate([out_s, out_v, pad], axis=1)


def _tc_edge(ea, sh, xd, w1, b1, gs, bs, gv, bv, rs, rv, sel, q3, p3):
    n_blk = E_PAD // BE
    full = lambda r, c: pl.BlockSpec((r, c), lambda i: (0, 0))
    out = pl.pallas_call(
        _tc_edge_body,
        grid=(n_blk,),
        in_specs=[
            pl.BlockSpec((BE, N_EDGE_FEAT), lambda i: (i, 0)),
            pl.BlockSpec((BE, SH_DIM), lambda i: (i, 0)),
            pl.BlockSpec((BE, 128), lambda i: (i, 0)),
            full(N_EDGE_FEAT, HIDDEN),
            full(8, HIDDEN),
            full(HIDDEN, 3072),
            full(8, 3072),
            full(HIDDEN, 1024),
            full(8, 1024),
            full(64, 3072),
            full(64, 1024),
            full(MUL_S, MUL_V),
            full(MUL_S, MUL_S),
            full(MUL_S, MUL_S),
        ],
        out_specs=pl.BlockSpec((BE, 128), lambda i: (i, 0)),
        out_shape=jax.ShapeDtypeStruct((E_PAD, 128), jnp.float32),
    )
    return out(ea, sh, xd, w1, b1, gs, bs, gv, bv, rs, rv, sel, q3, p3)


def _tc_ln_body(p0_ref, p1_ref, na_ref, lnc_ref, m2_ref, o_ref):
    psum = p0_ref[...] + p1_ref[...]
    ssum = psum[:, :NODE_DIM]
    cnt = psum[:, NODE_DIM:NODE_DIM + 1]
    x = ssum / jnp.maximum(cnt, 1.0) + na_ref[...]
    lnc = lnc_ref[...]
    w_s = lnc[0:1, :]
    b_s = lnc[1:2, :]
    ms_s = lnc[2:3, :]
    w_v = lnc[3:4, :]
    ms_v = lnc[4:5, :]
    f1 = x[:, :MUL_S]
    m1 = jnp.mean(f1, axis=1, keepdims=True)
    f1 = f1 - m1 * ms_s
    n1 = jnp.mean(f1 * f1, axis=1, keepdims=True)
    f1 = f1 * (lax.rsqrt(n1 + EPS) * w_s) + b_s
    x2 = x[:, MUL_S:]
    m2f = jnp.dot(x2, m2_ref[...])
    f2 = x2 - m2f * ms_v
    n2 = jnp.mean(f2 * f2, axis=1, keepdims=True)
    f2 = f2 * (lax.rsqrt(n2 + EPS) * w_v)
    o_ref[...] = jnp.concatenate([f1, f2], axis=1)


def _tc_ln(p0, p1, na_pad, lnc, m2c):
    n_blk = N_ACC // BN
    out = pl.pallas_call(
        _tc_ln_body,
        grid=(n_blk,),
        in_specs=[
            pl.BlockSpec((BN, 128), lambda i: (i, 0)),
            pl.BlockSpec((BN, 128), lambda i: (i, 0)),
            pl.BlockSpec((BN, NODE_DIM), lambda i: (i, 0)),
            pl.BlockSpec((8, MUL_S), lambda i: (0, 0)),
            pl.BlockSpec((MUL_S, MUL_S), lambda i: (0, 0)),
        ],
        out_specs=pl.BlockSpec((BN, NODE_DIM), lambda i: (i, 0)),
        out_shape=jax.ShapeDtypeStruct((N_ACC, NODE_DIM), jnp.float32),
    )
    return out(p0, p1, na_pad, lnc, m2c)


def _pad8(row):
    return np.pad(row[None, :], ((0, 7), (0, 0))).astype(np.float32)


_R_S = np.kron(np.eye(64), np.ones((1, MUL_S))).astype(np.float32)
_S_S = np.kron(np.ones((64, 1)), np.eye(MUL_S)).astype(np.float32)
_R_V = np.kron(np.eye(64), np.ones((1, MUL_V))).astype(np.float32)
_S_V = np.kron(np.ones((64, 1)), np.eye(MUL_V)).astype(np.float32)
_SEL = np.kron(np.eye(MUL_V), np.ones((3, 1))).astype(np.float32)
_Q3 = np.zeros((MUL_S, MUL_S), np.float32)
for _u in range(MUL_V):
    for _j in range(3):
        _Q3[3 * _u + _j, _j * MUL_V + _u] = 1.0
_P3 = np.zeros((MUL_S, MUL_S), np.float32)
for _w in range(MUL_V):
    for _j in range(3):
        _P3[_j * MUL_V + _w, 3 * _w + _j] = 1.0
_M2 = np.kron(np.ones((MUL_V, MUL_V), np.float32) / MUL_V,
              np.eye(3, dtype=np.float32))


def kernel(node_attr, edge_index, edge_attr, edge_sh, fc_w1, fc_b1, fc_w2,
           fc_b2, ln_weight, ln_bias, ln_mean_shift):
    f32 = jnp.float32
    edge_dst = jnp.concatenate(
        [edge_index[1], jnp.zeros((E_PAD - N_EDGES,), jnp.int32)])
    edge_src = jnp.concatenate(
        [edge_index[0], jnp.full((E_PAD - N_EDGES,), N_NODES, jnp.int32)])
    dst2d = edge_dst.reshape(NW, NCH, CHUNK)
    src2d = edge_src.reshape(NW, NCH, CHUNK)
    ea_pad = jnp.concatenate(
        [edge_attr, jnp.zeros((E_PAD - N_EDGES, N_EDGE_FEAT), f32)])
    sh_pad = jnp.concatenate(
        [edge_sh, jnp.zeros((E_PAD - N_EDGES, SH_DIM), f32)])
    na_pad = jnp.concatenate(
        [node_attr, jnp.zeros((N_ACC - N_NODES, NODE_DIM), f32)])
    na128 = jnp.concatenate(
        [node_attr, jnp.zeros((N_NODES, 128 - NODE_DIM), f32)], axis=1)

    bf16 = jnp.bfloat16
    g_s = jnp.concatenate([fc_w2[:, :2304], fc_w2[:, 3328:]],
                          axis=1).astype(bf16)
    g_v = fc_w2[:, 2304:3328].astype(bf16)
    b_s = jnp.tile(jnp.concatenate([fc_b2[:2304], fc_b2[3328:]])[None, :],
                   (8, 1)).astype(bf16)
    b_v = jnp.tile(fc_b2[2304:3328][None, :], (8, 1)).astype(bf16)
    b1r = jnp.tile(fc_b1[None, :], (8, 1))

    lnc = jnp.zeros((8, MUL_S), f32)
    lnc = lnc.at[0].set(ln_weight[:MUL_S])
    lnc = lnc.at[1].set(ln_bias)
    lnc = lnc.at[2].set(ln_mean_shift[0, :MUL_S, 0])
    lnc = lnc.at[3].set(jnp.repeat(ln_weight[MUL_S:], 3))
    lnc = lnc.at[4].set(jnp.repeat(ln_mean_shift[0, MUL_S:, 0], 3))

    z128 = jnp.zeros((NPW, 128), f32)

    x_dst = _sc_gather(na128, dst2d)
    tp = _tc_edge(ea_pad.astype(bf16), sh_pad, x_dst, fc_w1.astype(bf16),
                  b1r, g_s, b_s, g_v, b_v,
                  jnp.asarray(_R_S, bf16), jnp.asarray(_R_V, bf16),
                  jnp.asarray(_SEL), jnp.asarray(_Q3), jnp.asarray(_P3))
    summed = _sc_scatter(tp, src2d, z128)
    summed = summed.reshape(NC, N_ACC, 128)
    out = _tc_ln(summed[0], summed[1], na_pad, lnc, jnp.asarray(_M2))
    return out[:N_NODES]


# speedup vs baseline: 3.6002x; 1.0409x over previous
import functools

import numpy as np
import jax
import jax.numpy as jnp
from jax import lax
from jax.experimental import pallas as pl
from jax.experimental.pallas import tpu as pltpu
from jax.experimental.pallas import tpu_sc as plsc

N_NODES = 10000
N_EDGES = 40000
MUL_S = 48
MUL_V = 16
NODE_DIM = 96
SH_DIM = 4
N_EDGE_FEAT = 128
HIDDEN = 128
C_PATH = 0.125
EPS = 1e-5

NC = 2
NS = 16
NW = NC * NS
E_PAD = 40960
EPW = E_PAD // NW
CHUNK = 128
NCH = EPW // CHUNK
N_ACC = 10240
NPW = N_ACC // NS

BE = 512
BN = 512


def _sc_gather_body(table_hbm, idx_hbm, out_hbm, idx_v, rows_v, sem0, sem1):
    wid = lax.axis_index("s") * NC + lax.axis_index("c")
    sems = (sem0, sem1)
    pltpu.sync_copy(idx_hbm.at[wid], idx_v)
    cp = pltpu.async_copy(table_hbm.at[idx_v.at[0]], rows_v.at[0], sems[0])
    for j in range(NCH):
        cp.wait()
        if j + 1 < NCH:
            cp = pltpu.async_copy(table_hbm.at[idx_v.at[j + 1]],
                                  rows_v.at[(j + 1) % 2], sems[(j + 1) % 2])
        pltpu.sync_copy(rows_v.at[j % 2],
                        out_hbm.at[pl.ds(wid * EPW + j * CHUNK, CHUNK)])


def _sc_gather(node_attr_pad, idx3d):
    fn = pl.kernel(
        _sc_gather_body,
        out_type=jax.ShapeDtypeStruct((E_PAD, 128), jnp.float32),
        mesh=plsc.VectorSubcoreMesh(
            core_axis_name="c", subcore_axis_name="s", num_cores=NC,
            num_subcores=NS,
        ),
        scratch_types=[
            pltpu.VMEM((NCH, CHUNK), jnp.int32),
            pltpu.VMEM((2, CHUNK, 128), jnp.float32),
            pltpu.SemaphoreType.DMA,
            pltpu.SemaphoreType.DMA,
        ],
    )
    return fn(node_attr_pad, idx3d)


def _sc_scatter_body(tp_hbm, idx_hbm, z_hbm, out_hbm, idx_v, rows_v, acc_sp,
                     sem0, sem1):
    c = lax.axis_index("c")
    s = lax.axis_index("s")
    wid = s * NC + c
    sems = (sem0, sem1)
    pltpu.sync_copy(z_hbm, acc_sp.at[pl.ds(s * NPW, NPW)])
    pltpu.sync_copy(idx_hbm.at[wid], idx_v)
    cp = pltpu.async_copy(tp_hbm.at[pl.ds(wid * EPW, CHUNK)], rows_v.at[0],
                          sems[0])
    plsc.subcore_barrier()
    for j in range(NCH):
        cp.wait()
        if j + 1 < NCH:
            cp = pltpu.async_copy(
                tp_hbm.at[pl.ds(wid * EPW + (j + 1) * CHUNK, CHUNK)],
                rows_v.at[(j + 1) % 2], sems[(j + 1) % 2])
        pltpu.sync_copy(rows_v.at[j % 2], acc_sp.at[idx_v.at[j]], add=True)
    plsc.subcore_barrier()
    base = c * N_ACC + s * NPW
    pltpu.sync_copy(acc_sp.at[pl.ds(s * NPW, NPW)],
                    out_hbm.at[pl.ds(base, NPW)])


def _sc_scatter(tp, idx3d, z128):
    fn = pl.kernel(
        _sc_scatter_body,
        out_type=jax.ShapeDtypeStruct((NC * N_ACC, 128), jnp.float32),
        mesh=plsc.VectorSubcoreMesh(
            core_axis_name="c", subcore_axis_name="s", num_cores=NC,
            num_subcores=NS,
        ),
        scratch_types=[
            pltpu.VMEM((NCH, CHUNK), jnp.int32),
            pltpu.VMEM((2, CHUNK, 128), jnp.float32),
            pltpu.VMEM_SHARED((N_ACC, 128), jnp.float32),
            pltpu.SemaphoreType.DMA,
            pltpu.SemaphoreType.DMA,
        ],
    )
    return fn(tp, idx3d, z128)


def _fold_to(p, target):
    w = p.shape[1]
    while w > target:
        w //= 2
        p = p[:, :w] + p[:, w:2 * w]
    return p


def _tc_edge_body(ea_ref, sh_ref, xd_ref, w1_ref, b1_ref, gs_ref, bs_ref,
                  gv_ref, bv_ref, rs_ref, rv_ref, sel_ref,
                  q3_ref, sf_ref, o_ref):
    f32 = jnp.float32
    bf16 = jnp.bfloat16
    ea = ea_ref[...]
    sh = sh_ref[...]
    xd = xd_ref[:, :NODE_DIM]
    h = jax.nn.relu(
        jnp.dot(ea, w1_ref[...], preferred_element_type=f32)
        + b1_ref[0:1, :])
    hb = h.astype(bf16)
    ws = (jnp.dot(hb, gs_ref[...], preferred_element_type=f32)
          + bs_ref[0:1, :])
    wv = (jnp.dot(hb, gv_ref[...], preferred_element_type=f32)
          + bv_ref[0:1, :])
    xs = xd[:, :MUL_S]
    xvf = xd[:, MUL_S:]
    shs = sh[:, 0:1]
    shv = sh[:, 1:4]
    vv = jnp.concatenate([shv] * MUL_V, axis=1)
    bb = jnp.dot(xvf * vv, sel_ref[...])
    a_s = jnp.concatenate([xs * shs, bb], axis=1).astype(bf16)
    u_s = jnp.dot(a_s, rs_ref[...], preferred_element_type=f32)
    parts = [_fold_to(ws * u_s, 384)]
    xvp = jnp.dot(xvf, q3_ref[...])
    for j in range(3):
        a_vj = jnp.concatenate(
            [xs * shv[:, j:j + 1],
             xvp[:, j * MUL_V:(j + 1) * MUL_V] * shs],
            axis=1).astype(bf16)
        u_vj = jnp.dot(a_vj, rv_ref[...], preferred_element_type=f32)
        parts.append(_fold_to(wv * u_vj, 128))
    big = jnp.concatenate(parts, axis=1).astype(bf16)
    out = C_PATH * jnp.dot(big, sf_ref[...],
                           preferred_element_type=f32)
    n = out.shape[0]
    pad = jnp.concatenate(
        [jnp.ones((n, 1), jnp.float32), jnp.zeros((n, 31), jnp.float32)],
        axis=1)
    o_ref[...] = jnp.concatenate([out, pad], axis=1)


def _tc_edge(ea, sh, xd, w1, b1, gs, bs, gv, bv, rs, rv, sel, q3, sf):
    n_blk = E_PAD // BE
    full = lambda r, c: pl.BlockSpec((r, c), lambda i: (0, 0))
    out = pl.pallas_call(
        _tc_edge_body,
        grid=(n_blk,),
        in_specs=[
            pl.BlockSpec((BE, N_EDGE_FEAT), lambda i: (i, 0)),
            pl.BlockSpec((BE, SH_DIM), lambda i: (i, 0)),
            pl.BlockSpec((BE, 128), lambda i: (i, 0)),
            full(N_EDGE_FEAT, HIDDEN),
            full(8, HIDDEN),
            full(HIDDEN, 3072),
            full(8, 3072),
            full(HIDDEN, 1024),
            full(8, 1024),
            full(64, 3072),
            full(64, 1024),
            full(MUL_S, MUL_V),
            full(MUL_S, MUL_S),
            full(768, NODE_DIM),
        ],
        out_specs=pl.BlockSpec((BE, 128), lambda i: (i, 0)),
        out_shape=jax.ShapeDtypeStruct((E_PAD, 128), jnp.float32),
    )
    return out(ea, sh, xd, w1, b1, gs, bs, gv, bv, rs, rv, sel, q3, sf)


def _tc_ln_body(p0_ref, p1_ref, na_ref, lnc_ref, m2_ref, o_ref):
    psum = p0_ref[...] + p1_ref[...]
    ssum = psum[:, :NODE_DIM]
    cnt = psum[:, NODE_DIM:NODE_DIM + 1]
    x = ssum / jnp.maximum(cnt, 1.0) + na_ref[...]
    lnc = lnc_ref[...]
    w_s = lnc[0:1, :]
    b_s = lnc[1:2, :]
    ms_s = lnc[2:3, :]
    w_v = lnc[3:4, :]
    ms_v = lnc[4:5, :]
    f1 = x[:, :MUL_S]
    m1 = jnp.mean(f1, axis=1, keepdims=True)
    f1 = f1 - m1 * ms_s
    n1 = jnp.mean(f1 * f1, axis=1, keepdims=True)
    f1 = f1 * (lax.rsqrt(n1 + EPS) * w_s) + b_s
    x2 = x[:, MUL_S:]
    m2f = jnp.dot(x2, m2_ref[...])
    f2 = x2 - m2f * ms_v
    n2 = jnp.mean(f2 * f2, axis=1, keepdims=True)
    f2 = f2 * (lax.rsqrt(n2 + EPS) * w_v)
    o_ref[...] = jnp.concatenate([f1, f2], axis=1)


def _tc_ln(p0, p1, na_pad, lnc, m2c):
    n_blk = N_ACC // BN
    out = pl.pallas_call(
        _tc_ln_body,
        grid=(n_blk,),
        in_specs=[
            pl.BlockSpec((BN, 128), lambda i: (i, 0)),
            pl.BlockSpec((BN, 128), lambda i: (i, 0)),
            pl.BlockSpec((BN, NODE_DIM), lambda i: (i, 0)),
            pl.BlockSpec((8, MUL_S), lambda i: (0, 0)),
            pl.BlockSpec((MUL_S, MUL_S), lambda i: (0, 0)),
        ],
        out_specs=pl.BlockSpec((BN, NODE_DIM), lambda i: (i, 0)),
        out_shape=jax.ShapeDtypeStruct((N_ACC, NODE_DIM), jnp.float32),
    )
    return out(p0, p1, na_pad, lnc, m2c)


def _pad8(row):
    return np.pad(row[None, :], ((0, 7), (0, 0))).astype(np.float32)


_R_S = np.kron(np.eye(64), np.ones((1, MUL_S))).astype(np.float32)
_S_S = np.kron(np.ones((64, 1)), np.eye(MUL_S)).astype(np.float32)
_R_V = np.kron(np.eye(64), np.ones((1, MUL_V))).astype(np.float32)
_S_V = np.kron(np.ones((64, 1)), np.eye(MUL_V)).astype(np.float32)
_SEL = np.kron(np.eye(MUL_V), np.ones((3, 1))).astype(np.float32)
_Q3 = np.zeros((MUL_S, MUL_S), np.float32)
for _u in range(MUL_V):
    for _j in range(3):
        _Q3[3 * _u + _j, _j * MUL_V + _u] = 1.0
_SF = np.zeros((768, NODE_DIM), np.float32)
_SF[:384, :MUL_S] = np.kron(np.ones((8, 1)), np.eye(MUL_S))
for _j in range(3):
    _pj = np.zeros((MUL_V, MUL_S), np.float32)
    for _w in range(MUL_V):
        _pj[_w, 3 * _w + _j] = 1.0
    _SF[384 + _j * 128:384 + (_j + 1) * 128, MUL_S:] = np.kron(
        np.ones((8, 1)), _pj)
_M2 = np.kron(np.ones((MUL_V, MUL_V), np.float32) / MUL_V,
              np.eye(3, dtype=np.float32))


def kernel(node_attr, edge_index, edge_attr, edge_sh, fc_w1, fc_b1, fc_w2,
           fc_b2, ln_weight, ln_bias, ln_mean_shift):
    f32 = jnp.float32
    edge_dst = jnp.concatenate(
        [edge_index[1], jnp.zeros((E_PAD - N_EDGES,), jnp.int32)])
    edge_src = jnp.concatenate(
        [edge_index[0], jnp.full((E_PAD - N_EDGES,), N_NODES, jnp.int32)])
    dst2d = edge_dst.reshape(NW, NCH, CHUNK)
    src2d = edge_src.reshape(NW, NCH, CHUNK)
    ea_pad = jnp.concatenate(
        [edge_attr, jnp.zeros((E_PAD - N_EDGES, N_EDGE_FEAT), f32)])
    sh_pad = jnp.concatenate(
        [edge_sh, jnp.zeros((E_PAD - N_EDGES, SH_DIM), f32)])
    na_pad = jnp.concatenate(
        [node_attr, jnp.zeros((N_ACC - N_NODES, NODE_DIM), f32)])
    na128 = jnp.concatenate(
        [node_attr, jnp.zeros((N_NODES, 128 - NODE_DIM), f32)], axis=1)

    bf16 = jnp.bfloat16
    g_s = jnp.concatenate([fc_w2[:, :2304], fc_w2[:, 3328:]],
                          axis=1).astype(bf16)
    g_v = fc_w2[:, 2304:3328].astype(bf16)
    b_s = jnp.tile(jnp.concatenate([fc_b2[:2304], fc_b2[3328:]])[None, :],
                   (8, 1)).astype(bf16)
    b_v = jnp.tile(fc_b2[2304:3328][None, :], (8, 1)).astype(bf16)
    b1r = jnp.tile(fc_b1[None, :], (8, 1))

    lnc = jnp.zeros((8, MUL_S), f32)
    lnc = lnc.at[0].set(ln_weight[:MUL_S])
    lnc = lnc.at[1].set(ln_bias)
    lnc = lnc.at[2].set(ln_mean_shift[0, :MUL_S, 0])
    lnc = lnc.at[3].set(jnp.repeat(ln_weight[MUL_S:], 3))
    lnc = lnc.at[4].set(jnp.repeat(ln_mean_shift[0, MUL_S:, 0], 3))

    z128 = jnp.zeros((NPW, 128), f32)

    x_dst = _sc_gather(na128, dst2d)
    tp = _tc_edge(ea_pad.astype(bf16), sh_pad, x_dst, fc_w1.astype(bf16),
                  b1r, g_s, b_s, g_v, b_v,
                  jnp.asarray(_R_S, bf16), jnp.asarray(_R_V, bf16),
                  jnp.asarray(_SEL), jnp.asarray(_Q3),
                  jnp.asarray(_SF, bf16))
    summed = _sc_scatter(tp, src2d, z128)
    summed = summed.reshape(NC, N_ACC, 128)
    out = _tc_ln(summed[0], summed[1], na_pad, lnc, jnp.asarray(_M2))
    return out[:N_NODES]


# speedup vs baseline: 3.7651x; 1.0458x over previous
import functools

import numpy as np
import jax
import jax.numpy as jnp
from jax import lax
from jax.experimental import pallas as pl
from jax.experimental.pallas import tpu as pltpu
from jax.experimental.pallas import tpu_sc as plsc

N_NODES = 10000
N_EDGES = 40000
MUL_S = 48
MUL_V = 16
NODE_DIM = 96
SH_DIM = 4
N_EDGE_FEAT = 128
HIDDEN = 128
C_PATH = 0.125
EPS = 1e-5

NC = 2
NS = 16
NW = NC * NS
E_PAD = 40960
EPW = E_PAD // NW
CHUNK = 128
NCH = EPW // CHUNK
N_ACC = 10240
NPW = N_ACC // NS

BE = 1024
BN = 512


def _sc_gather_body(table_hbm, idx_hbm, out_hbm, idx_v, rows_v, sem0, sem1):
    wid = lax.axis_index("s") * NC + lax.axis_index("c")
    sems = (sem0, sem1)
    pltpu.sync_copy(idx_hbm.at[wid], idx_v)
    cp = pltpu.async_copy(table_hbm.at[idx_v.at[0]], rows_v.at[0], sems[0])
    for j in range(NCH):
        cp.wait()
        if j + 1 < NCH:
            cp = pltpu.async_copy(table_hbm.at[idx_v.at[j + 1]],
                                  rows_v.at[(j + 1) % 2], sems[(j + 1) % 2])
        pltpu.sync_copy(rows_v.at[j % 2],
                        out_hbm.at[pl.ds(wid * EPW + j * CHUNK, CHUNK)])


def _sc_gather(node_attr_pad, idx3d):
    fn = pl.kernel(
        _sc_gather_body,
        out_type=jax.ShapeDtypeStruct((E_PAD, 128), jnp.float32),
        mesh=plsc.VectorSubcoreMesh(
            core_axis_name="c", subcore_axis_name="s", num_cores=NC,
            num_subcores=NS,
        ),
        scratch_types=[
            pltpu.VMEM((NCH, CHUNK), jnp.int32),
            pltpu.VMEM((2, CHUNK, 128), jnp.float32),
            pltpu.SemaphoreType.DMA,
            pltpu.SemaphoreType.DMA,
        ],
    )
    return fn(node_attr_pad, idx3d)


def _sc_scatter_body(tp_hbm, idx_hbm, z_hbm, out_hbm, idx_v, rows_v, acc_sp,
                     sem0, sem1):
    c = lax.axis_index("c")
    s = lax.axis_index("s")
    wid = s * NC + c
    sems = (sem0, sem1)
    pltpu.sync_copy(z_hbm, acc_sp.at[pl.ds(s * NPW, NPW)])
    pltpu.sync_copy(idx_hbm.at[wid], idx_v)
    cp = pltpu.async_copy(tp_hbm.at[pl.ds(wid * EPW, CHUNK)], rows_v.at[0],
                          sems[0])
    plsc.subcore_barrier()
    for j in range(NCH):
        cp.wait()
        if j + 1 < NCH:
            cp = pltpu.async_copy(
                tp_hbm.at[pl.ds(wid * EPW + (j + 1) * CHUNK, CHUNK)],
                rows_v.at[(j + 1) % 2], sems[(j + 1) % 2])
        pltpu.sync_copy(rows_v.at[j % 2], acc_sp.at[idx_v.at[j]], add=True)
    plsc.subcore_barrier()
    base = c * N_ACC + s * NPW
    pltpu.sync_copy(acc_sp.at[pl.ds(s * NPW, NPW)],
                    out_hbm.at[pl.ds(base, NPW)])


def _sc_scatter(tp, idx3d, z128):
    fn = pl.kernel(
        _sc_scatter_body,
        out_type=jax.ShapeDtypeStruct((NC * N_ACC, 128), jnp.float32),
        mesh=plsc.VectorSubcoreMesh(
            core_axis_name="c", subcore_axis_name="s", num_cores=NC,
            num_subcores=NS,
        ),
        scratch_types=[
            pltpu.VMEM((NCH, CHUNK), jnp.int32),
            pltpu.VMEM((2, CHUNK, 128), jnp.float32),
            pltpu.VMEM_SHARED((N_ACC, 128), jnp.float32),
            pltpu.SemaphoreType.DMA,
            pltpu.SemaphoreType.DMA,
        ],
    )
    return fn(tp, idx3d, z128)


def _fold_to(p, target):
    w = p.shape[1]
    while w > target:
        w //= 2
        p = p[:, :w] + p[:, w:2 * w]
    return p


def _tc_edge_body(ea_ref, sh_ref, xd_ref, w1_ref, b1_ref, gs_ref, bs_ref,
                  gv_ref, bv_ref, rs_ref, rv_ref, sel_ref,
                  q3_ref, sf_ref, o_ref):
    f32 = jnp.float32
    bf16 = jnp.bfloat16
    ea = ea_ref[...]
    sh = sh_ref[...]
    xd = xd_ref[:, :NODE_DIM]
    h = jax.nn.relu(
        jnp.dot(ea, w1_ref[...], preferred_element_type=f32)
        + b1_ref[0:1, :])
    hb = h.astype(bf16)
    ws = (jnp.dot(hb, gs_ref[...], preferred_element_type=f32)
          + bs_ref[0:1, :])
    wv = (jnp.dot(hb, gv_ref[...], preferred_element_type=f32)
          + bv_ref[0:1, :])
    xs = xd[:, :MUL_S]
    xvf = xd[:, MUL_S:]
    shs = sh[:, 0:1]
    shv = sh[:, 1:4]
    vv = jnp.concatenate([shv] * MUL_V, axis=1)
    bb = jnp.dot(xvf * vv, sel_ref[...])
    a_s = jnp.concatenate([xs * shs, bb], axis=1).astype(bf16)
    u_s = jnp.dot(a_s, rs_ref[...], preferred_element_type=f32)
    parts = [_fold_to(ws * u_s, 384)]
    xvp = jnp.dot(xvf, q3_ref[...])
    for j in range(3):
        a_vj = jnp.concatenate(
            [xs * shv[:, j:j + 1],
             xvp[:, j * MUL_V:(j + 1) * MUL_V] * shs],
            axis=1).astype(bf16)
        u_vj = jnp.dot(a_vj, rv_ref[...], preferred_element_type=f32)
        parts.append(_fold_to(wv * u_vj, 128))
    big = jnp.concatenate(parts, axis=1).astype(bf16)
    out = C_PATH * jnp.dot(big, sf_ref[...],
                           preferred_element_type=f32)
    n = out.shape[0]
    pad = jnp.concatenate(
        [jnp.ones((n, 1), jnp.float32), jnp.zeros((n, 31), jnp.float32)],
        axis=1)
    o_ref[...] = jnp.concatenate([out, pad], axis=1)


def _tc_edge(ea, sh, xd, w1, b1, gs, bs, gv, bv, rs, rv, sel, q3, sf):
    n_blk = E_PAD // BE
    full = lambda r, c: pl.BlockSpec((r, c), lambda i: (0, 0))
    out = pl.pallas_call(
        _tc_edge_body,
        grid=(n_blk,),
        in_specs=[
            pl.BlockSpec((BE, N_EDGE_FEAT), lambda i: (i, 0)),
            pl.BlockSpec((BE, SH_DIM), lambda i: (i, 0)),
            pl.BlockSpec((BE, 128), lambda i: (i, 0)),
            full(N_EDGE_FEAT, HIDDEN),
            full(8, HIDDEN),
            full(HIDDEN, 3072),
            full(8, 3072),
            full(HIDDEN, 1024),
            full(8, 1024),
            full(64, 3072),
            full(64, 1024),
            full(MUL_S, MUL_V),
            full(MUL_S, MUL_S),
            full(768, NODE_DIM),
        ],
        out_specs=pl.BlockSpec((BE, 128), lambda i: (i, 0)),
        out_shape=jax.ShapeDtypeStruct((E_PAD, 128), jnp.float32),
    )
    return out(ea, sh, xd, w1, b1, gs, bs, gv, bv, rs, rv, sel, q3, sf)


def _tc_ln_body(p0_ref, p1_ref, na_ref, lnc_ref, m2_ref, o_ref):
    psum = p0_ref[...] + p1_ref[...]
    ssum = psum[:, :NODE_DIM]
    cnt = psum[:, NODE_DIM:NODE_DIM + 1]
    x = ssum / jnp.maximum(cnt, 1.0) + na_ref[...]
    lnc = lnc_ref[...]
    w_s = lnc[0:1, :]
    b_s = lnc[1:2, :]
    ms_s = lnc[2:3, :]
    w_v = lnc[3:4, :]
    ms_v = lnc[4:5, :]
    f1 = x[:, :MUL_S]
    m1 = jnp.mean(f1, axis=1, keepdims=True)
    f1 = f1 - m1 * ms_s
    n1 = jnp.mean(f1 * f1, axis=1, keepdims=True)
    f1 = f1 * (lax.rsqrt(n1 + EPS) * w_s) + b_s
    x2 = x[:, MUL_S:]
    m2f = jnp.dot(x2, m2_ref[...])
    f2 = x2 - m2f * ms_v
    n2 = jnp.mean(f2 * f2, axis=1, keepdims=True)
    f2 = f2 * (lax.rsqrt(n2 + EPS) * w_v)
    o_ref[...] = jnp.concatenate([f1, f2], axis=1)


def _tc_ln(p0, p1, na_pad, lnc, m2c):
    n_blk = N_ACC // BN
    out = pl.pallas_call(
        _tc_ln_body,
        grid=(n_blk,),
        in_specs=[
            pl.BlockSpec((BN, 128), lambda i: (i, 0)),
            pl.BlockSpec((BN, 128), lambda i: (i, 0)),
            pl.BlockSpec((BN, NODE_DIM), lambda i: (i, 0)),
            pl.BlockSpec((8, MUL_S), lambda i: (0, 0)),
            pl.BlockSpec((MUL_S, MUL_S), lambda i: (0, 0)),
        ],
        out_specs=pl.BlockSpec((BN, NODE_DIM), lambda i: (i, 0)),
        out_shape=jax.ShapeDtypeStruct((N_ACC, NODE_DIM), jnp.float32),
    )
    return out(p0, p1, na_pad, lnc, m2c)


def _pad8(row):
    return np.pad(row[None, :], ((0, 7), (0, 0))).astype(np.float32)


_R_S = np.kron(np.eye(64), np.ones((1, MUL_S))).astype(np.float32)
_S_S = np.kron(np.ones((64, 1)), np.eye(MUL_S)).astype(np.float32)
_R_V = np.kron(np.eye(64), np.ones((1, MUL_V))).astype(np.float32)
_S_V = np.kron(np.ones((64, 1)), np.eye(MUL_V)).astype(np.float32)
_SEL = np.kron(np.eye(MUL_V), np.ones((3, 1))).astype(np.float32)
_Q3 = np.zeros((MUL_S, MUL_S), np.float32)
for _u in range(MUL_V):
    for _j in range(3):
        _Q3[3 * _u + _j, _j * MUL_V + _u] = 1.0
_SF = np.zeros((768, NODE_DIM), np.float32)
_SF[:384, :MUL_S] = np.kron(np.ones((8, 1)), np.eye(MUL_S))
for _j in range(3):
    _pj = np.zeros((MUL_V, MUL_S), np.float32)
    for _w in range(MUL_V):
        _pj[_w, 3 * _w + _j] = 1.0
    _SF[384 + _j * 128:384 + (_j + 1) * 128, MUL_S:] = np.kron(
        np.ones((8, 1)), _pj)
_M2 = np.kron(np.ones((MUL_V, MUL_V), np.float32) / MUL_V,
              np.eye(3, dtype=np.float32))


def kernel(node_attr, edge_index, edge_attr, edge_sh, fc_w1, fc_b1, fc_w2,
           fc_b2, ln_weight, ln_bias, ln_mean_shift):
    f32 = jnp.float32
    edge_dst = jnp.concatenate(
        [edge_index[1], jnp.zeros((E_PAD - N_EDGES,), jnp.int32)])
    edge_src = jnp.concatenate(
        [edge_index[0], jnp.full((E_PAD - N_EDGES,), N_NODES, jnp.int32)])
    dst2d = edge_dst.reshape(NW, NCH, CHUNK)
    src2d = edge_src.reshape(NW, NCH, CHUNK)
    ea_pad = jnp.concatenate(
        [edge_attr, jnp.zeros((E_PAD - N_EDGES, N_EDGE_FEAT), f32)])
    sh_pad = jnp.concatenate(
        [edge_sh, jnp.zeros((E_PAD - N_EDGES, SH_DIM), f32)])
    na_pad = jnp.concatenate(
        [node_attr, jnp.zeros((N_ACC - N_NODES, NODE_DIM), f32)])
    na128 = jnp.concatenate(
        [node_attr, jnp.zeros((N_NODES, 128 - NODE_DIM), f32)], axis=1)

    bf16 = jnp.bfloat16
    g_s = jnp.concatenate([fc_w2[:, :2304], fc_w2[:, 3328:]],
                          axis=1).astype(bf16)
    g_v = fc_w2[:, 2304:3328].astype(bf16)
    b_s = jnp.tile(jnp.concatenate([fc_b2[:2304], fc_b2[3328:]])[None, :],
                   (8, 1)).astype(bf16)
    b_v = jnp.tile(fc_b2[2304:3328][None, :], (8, 1)).astype(bf16)
    b1r = jnp.tile(fc_b1[None, :], (8, 1))

    lnc = jnp.zeros((8, MUL_S), f32)
    lnc = lnc.at[0].set(ln_weight[:MUL_S])
    lnc = lnc.at[1].set(ln_bias)
    lnc = lnc.at[2].set(ln_mean_shift[0, :MUL_S, 0])
    lnc = lnc.at[3].set(jnp.repeat(ln_weight[MUL_S:], 3))
    lnc = lnc.at[4].set(jnp.repeat(ln_mean_shift[0, MUL_S:, 0], 3))

    z128 = jnp.zeros((NPW, 128), f32)

    x_dst = _sc_gather(na128, dst2d)
    tp = _tc_edge(ea_pad.astype(bf16), sh_pad, x_dst, fc_w1.astype(bf16),
                  b1r, g_s, b_s, g_v, b_v,
                  jnp.asarray(_R_S, bf16), jnp.asarray(_R_V, bf16),
                  jnp.asarray(_SEL), jnp.asarray(_Q3),
                  jnp.asarray(_SF, bf16))
    summed = _sc_scatter(tp, src2d, z128)
    summed = summed.reshape(NC, N_ACC, 128)
    out = _tc_ln(summed[0], summed[1], na_pad, lnc, jnp.asarray(_M2))
    return out[:N_NODES]
